# baseline (speedup 1.0000x reference)
import functools

import jax
import jax.numpy as jnp
from jax import lax
from jax.experimental import pallas as pl
from jax.experimental.pallas import tpu as pltpu
from jax.experimental.pallas import tpu_sc as plsc

_CHUNK = 128


@functools.cache
def _make_gather(n_rows, d, n_table):
    info = plsc.get_sparse_core_info()
    nc, ns = info.num_cores, info.num_subcores
    nw = nc * ns
    per_w = n_rows // nw
    n_chunks = per_w // _CHUNK
    t_per_s = n_table // ns
    mesh = plsc.VectorSubcoreMesh(core_axis_name="c", subcore_axis_name="s")

    @functools.partial(
        pl.kernel,
        mesh=mesh,
        out_type=jax.ShapeDtypeStruct((n_rows, d), jnp.float32),
        scratch_types=[
            pltpu.VMEM((n_chunks, _CHUNK), jnp.int32),
            pltpu.VMEM((3, _CHUNK, d), jnp.float32),
            pltpu.VMEM_SHARED((n_table, d), jnp.float32),
            pltpu.SemaphoreType.DMA((3,)),
            pltpu.SemaphoreType.DMA((3,)),
            pltpu.SemaphoreType.DMA,
        ],
    )
    def gather_kernel(table_hbm, idx_hbm, out_hbm, idx_v, rows_v, tab_sh,
                      gsem, wsem, tsem):
        sid = lax.axis_index("s")
        wid = sid * nc + lax.axis_index("c")
        base = wid * per_w
        stage = pltpu.make_async_copy(
            table_hbm.at[pl.ds(sid * t_per_s, t_per_s)],
            tab_sh.at[pl.ds(sid * t_per_s, t_per_s)],
            tsem,
        )
        stage.start()
        pltpu.sync_copy(idx_hbm.at[wid], idx_v)

        _K = 4

        def fire_gather(u, slot):
            from_hbm = u < _K + 2

            @pl.when(from_hbm)
            def _():
                pltpu.async_copy(
                    table_hbm.at[idx_v.at[u]], rows_v.at[slot], gsem.at[slot]
                )

            @pl.when(jnp.logical_not(from_hbm))
            def _():
                pltpu.async_copy(
                    tab_sh.at[idx_v.at[u]], rows_v.at[slot], gsem.at[slot]
                )

        def wait_gather(slot):
            pltpu.make_async_copy(
                out_hbm.at[pl.ds(0, _CHUNK)], rows_v.at[slot], gsem.at[slot]
            ).wait()

        def write_desc(u, slot):
            return pltpu.make_async_copy(
                rows_v.at[slot],
                out_hbm.at[pl.ds(base + u * _CHUNK, _CHUNK)],
                wsem.at[slot],
            )

        pltpu.async_copy(table_hbm.at[idx_v.at[0]], rows_v.at[0], gsem.at[0])
        pltpu.async_copy(table_hbm.at[idx_v.at[1]], rows_v.at[1], gsem.at[1])

        def body(u, _):
            slot = lax.rem(u, 3)

            @pl.when(u == _K)
            def _():
                stage.wait()
                plsc.subcore_barrier()

            @pl.when(u >= 1)
            def _():
                write_desc(u - 1, lax.rem(u - 1, 3)).wait()

            @pl.when(u + 2 < n_chunks)
            def _():
                fire_gather(u + 2, lax.rem(u + 2, 3))

            wait_gather(slot)
            write_desc(u, slot).start()
            return 0

        lax.fori_loop(0, n_chunks, body, 0)
        write_desc(n_chunks - 1, lax.rem(n_chunks - 1, 3)).wait()

    return gather_kernel


def kernel(pe, timesteps, index_select):
    if timesteps.ndim == 1:
        return pe[: timesteps.shape[0]]
    b, l = timesteps.shape
    n = b * l
    d = pe.shape[1]
    idx = (timesteps.reshape(-1) * index_select).astype(jnp.int32)
    info = plsc.get_sparse_core_info()
    nw = info.num_cores * info.num_subcores
    idx3d = idx.reshape(nw, n // (nw * _CHUNK), _CHUNK)
    out = _make_gather(n, d, pe.shape[0])(pe, idx3d)
    return out.reshape(b, l, d)


# speedup vs baseline: 1.0526x; 1.0526x over previous
import functools

import jax
import jax.numpy as jnp
from jax import lax
from jax.experimental import pallas as pl
from jax.experimental.pallas import tpu as pltpu
from jax.experimental.pallas import tpu_sc as plsc

_CHUNK = 128


@functools.cache
def _make_gather(n_rows, d, n_table):
    info = plsc.get_sparse_core_info()
    nc, ns = info.num_cores, info.num_subcores
    nw = nc * ns
    per_w = n_rows // nw
    n_chunks = per_w // _CHUNK
    t_per_s = n_table // ns
    mesh = plsc.VectorSubcoreMesh(core_axis_name="c", subcore_axis_name="s")

    @functools.partial(
        pl.kernel,
        mesh=mesh,
        out_type=jax.ShapeDtypeStruct((n_rows, d), jnp.float32),
        scratch_types=[
            pltpu.VMEM((n_chunks, _CHUNK), jnp.int32),
            pltpu.VMEM((3, _CHUNK, d), jnp.float32),
            pltpu.VMEM_SHARED((n_table, d), jnp.float32),
            pltpu.SemaphoreType.DMA((3,)),
            pltpu.SemaphoreType.DMA((3,)),
            pltpu.SemaphoreType.DMA,
        ],
    )
    def gather_kernel(table_hbm, idx_hbm, out_hbm, idx_v, rows_v, tab_sh,
                      gsem, wsem, tsem):
        sid = lax.axis_index("s")
        wid = sid * nc + lax.axis_index("c")
        base = wid * per_w
        stage = pltpu.make_async_copy(
            table_hbm.at[pl.ds(sid * t_per_s, t_per_s)],
            tab_sh.at[pl.ds(sid * t_per_s, t_per_s)],
            tsem,
        )
        stage.start()
        pltpu.sync_copy(idx_hbm.at[wid], idx_v)

        def fire_gather(u, slot):
            pltpu.async_copy(
                tab_sh.at[idx_v.at[u]], rows_v.at[slot], gsem.at[slot]
            )

        def wait_gather(slot):
            pltpu.make_async_copy(
                out_hbm.at[pl.ds(0, _CHUNK)], rows_v.at[slot], gsem.at[slot]
            ).wait()

        def write_desc(u, slot):
            return pltpu.make_async_copy(
                rows_v.at[slot],
                out_hbm.at[pl.ds(base + u * _CHUNK, _CHUNK)],
                wsem.at[slot],
            )

        pltpu.async_copy(table_hbm.at[idx_v.at[0]], rows_v.at[0], gsem.at[0])
        pltpu.async_copy(table_hbm.at[idx_v.at[1]], rows_v.at[1], gsem.at[1])
        stage.wait()
        plsc.subcore_barrier()

        def body(u, _):
            slot = lax.rem(u, 3)

            @pl.when(u >= 1)
            def _():
                write_desc(u - 1, lax.rem(u - 1, 3)).wait()

            @pl.when(u + 2 < n_chunks)
            def _():
                fire_gather(u + 2, lax.rem(u + 2, 3))

            wait_gather(slot)
            write_desc(u, slot).start()
            return 0

        lax.fori_loop(0, n_chunks, body, 0)
        write_desc(n_chunks - 1, lax.rem(n_chunks - 1, 3)).wait()

    return gather_kernel


def kernel(pe, timesteps, index_select):
    if timesteps.ndim == 1:
        return pe[: timesteps.shape[0]]
    b, l = timesteps.shape
    n = b * l
    d = pe.shape[1]
    idx = (timesteps.reshape(-1) * index_select).astype(jnp.int32)
    info = plsc.get_sparse_core_info()
    nw = info.num_cores * info.num_subcores
    idx3d = idx.reshape(nw, n // (nw * _CHUNK), _CHUNK)
    out = _make_gather(n, d, pe.shape[0])(pe, idx3d)
    return out.reshape(b, l, d)
